# baseline (speedup 1.0000x reference)
import jax
import jax.numpy as jnp
from jax.experimental import pallas as pl

_ITEM_NUM = 1000000
_LIST_LEN = 20
_R = 20000
_G = _ITEM_NUM // _R


def kernel(x, item_list):
    def body(a_ref, b_ref, o_ref):
        o_ref[0:_R - 2, :] = a_ref[2:_R, :]
        o_ref[_R - 2:_R, :] = b_ref[0:2, :]

    return pl.pallas_call(
        body,
        grid=(_G,),
        in_specs=[
            pl.BlockSpec((_R, _LIST_LEN), lambda i: (i, 0)),
            pl.BlockSpec((8, _LIST_LEN), lambda i: ((_R // 8) * (i + 1), 0)),
        ],
        out_specs=pl.BlockSpec((_R, _LIST_LEN), lambda i: (i, 0)),
        out_shape=jax.ShapeDtypeStruct((_ITEM_NUM, _LIST_LEN), jnp.float32),
    )(item_list, item_list)
